# baseline (speedup 1.0000x reference)
import jax
import jax.numpy as jnp
from jax import lax
from jax.experimental import pallas as pl
from jax.experimental.pallas import tpu as pltpu
from jax.experimental.pallas import tpu_sc as plsc

_NCOMP = 3
_GRID = 6
_ODIM = 128
_TN = 1024

_NC = 2
_NS = 16
_NW = _NC * _NS
_LANES = 16
_TB = 128
_OSTRIDE = 129
_REP = _NCOMP * _GRID * _ODIM + 1


def _stage_a_body(x_ref, p_ref, w_ref, off_ref, a_ref, b_ref):
    x = x_ref[...]
    projt = lax.dot_general(
        p_ref[...], x, (((0,), (1,)), ((), ())),
        preferred_element_type=jnp.float32,
    )
    p = jnp.clip(projt, -0.99, 0.99)
    xg = (p + 1.0) * (0.5 * (_GRID - 1))
    idx = jnp.clip(xg.astype(jnp.int32), 0, _GRID - 2)
    gs = idx.astype(jnp.float32) * (2.0 / (_GRID - 1)) - 1.0
    t = (p - gs) * (0.5 * (_GRID - 1))
    w = w_ref[...]
    comp = lax.broadcasted_iota(jnp.int32, idx.shape, 0)
    off_ref[...] = (comp * _GRID + idx) * _ODIM
    a_ref[...] = w * (1.0 - t)
    b_ref[...] = w * t


def _stage_a(x, pmat, w):
    n = x.shape[0]
    d = x.shape[1]
    shape = jax.ShapeDtypeStruct((_NCOMP, n), jnp.float32)
    return pl.pallas_call(
        _stage_a_body,
        grid=(n // _TN,),
        in_specs=[
            pl.BlockSpec((_TN, d), lambda i: (i, 0)),
            pl.BlockSpec((d, _NCOMP), lambda i: (0, 0)),
            pl.BlockSpec((_NCOMP, 1), lambda i: (0, 0)),
        ],
        out_specs=[
            pl.BlockSpec((_NCOMP, _TN), lambda i: (0, i)),
            pl.BlockSpec((_NCOMP, _TN), lambda i: (0, i)),
            pl.BlockSpec((_NCOMP, _TN), lambda i: (0, i)),
        ],
        out_shape=[
            jax.ShapeDtypeStruct((_NCOMP, n), jnp.int32),
            shape,
            shape,
        ],
        compiler_params=pltpu.CompilerParams(
            dimension_semantics=("arbitrary",)
        ),
    )(x, pmat, w)


def _tanh_sc(y):
    e = jnp.exp(-2.0 * jnp.abs(y))
    th = (1.0 - e) / (1.0 + e)
    return jnp.where(y < 0.0, -th, th)


def _stage_b_body(off_hbm, a_hbm, b_hbm, tbl_hbm, out_hbm,
                  offv, av, bv, tblv, outv):
    wid = lax.axis_index("s") * _NC + lax.axis_index("c")
    n = off_hbm.shape[0] // _NCOMP
    cn = n // _NW
    base = wid * cn
    for i in range(_NCOMP):
        pltpu.sync_copy(off_hbm.at[pl.ds(i * n + base, cn)],
                        offv.at[pl.ds(i * cn, cn)])
        pltpu.sync_copy(a_hbm.at[pl.ds(i * n + base, cn)],
                        av.at[pl.ds(i * cn, cn)])
        pltpu.sync_copy(b_hbm.at[pl.ds(i * n + base, cn)],
                        bv.at[pl.ds(i * cn, cn)])
    pltpu.sync_copy(tbl_hbm, tblv)

    nblk = cn // _TB
    ngrp = _TB // _LANES
    lane = lax.iota(jnp.int32, _LANES)
    lanebase = lane * _REP

    def grp_body(g, blk):
        gb = blk * _TB + g * _LANES
        offg = [offv[pl.ds(i * cn + gb, _LANES)] for i in range(_NCOMP)]
        ag = [av[pl.ds(i * cn + gb, _LANES)] for i in range(_NCOMP)]
        bg = [bv[pl.ds(i * cn + gb, _LANES)] for i in range(_NCOMP)]
        bases = [offg[i] + lanebase for i in range(_NCOMP)]
        rows = g * _LANES + lane

        for d in range(_ODIM):
            acc = None
            for i in range(_NCOMP):
                p0 = plsc.load_gather(tblv, [bases[i] + d])
                p1 = plsc.load_gather(tblv, [bases[i] + (_ODIM + d)])
                contrib = ag[i] * p0 + bg[i] * p1
                acc = contrib if i == 0 else acc + contrib
            cols = jnp.full((_LANES,), d, jnp.int32)
            plsc.store_scatter(outv, [rows, cols], _tanh_sc(acc))
        return blk

    def blk_body(blk, carry):
        lax.fori_loop(0, ngrp, grp_body, blk)
        pltpu.sync_copy(
            outv.at[:, pl.ds(0, _ODIM)],
            out_hbm.at[pl.ds(base + blk * _TB, _TB)],
        )
        return carry

    lax.fori_loop(0, nblk, blk_body, 0)


def _stage_b(offs, a, b, tbl):
    n = offs.shape[0] // _NCOMP
    cn = n // _NW
    body = pl.kernel(
        _stage_b_body,
        out_type=jax.ShapeDtypeStruct((n, _ODIM), jnp.float32),
        mesh=plsc.VectorSubcoreMesh(
            core_axis_name="c", subcore_axis_name="s"
        ),
        compiler_params=pltpu.CompilerParams(needs_layout_passes=False),
        scratch_types=[
            pltpu.VMEM((_NCOMP * cn,), jnp.int32),
            pltpu.VMEM((_NCOMP * cn,), jnp.float32),
            pltpu.VMEM((_NCOMP * cn,), jnp.float32),
            pltpu.VMEM((_LANES * _REP,), jnp.float32),
            pltpu.VMEM((_TB, _OSTRIDE), jnp.float32),
        ],
    )
    return body(offs, a, b, tbl)


def kernel(inputs, projections, control_points, component_weights):
    B, S, D = inputs.shape
    n = B * S
    x = inputs.reshape(n, D)
    pmat = projections[:, :, 0].T
    w = component_weights.reshape(_NCOMP, 1)
    flat = control_points.reshape(-1)
    rep = jnp.pad(flat, (0, _REP - flat.shape[0]))
    tbl = jnp.broadcast_to(rep, (_LANES, _REP)).reshape(-1)

    offs, a, b = _stage_a(x, pmat, w)
    out = _stage_b(offs.reshape(-1), a.reshape(-1), b.reshape(-1), tbl)
    return out.reshape(B, S, _ODIM)


# speedup vs baseline: 1.4332x; 1.4332x over previous
import jax
import jax.numpy as jnp
from jax import lax
from jax.experimental import pallas as pl
from jax.experimental.pallas import tpu as pltpu
from jax.experimental.pallas import tpu_sc as plsc

_NCOMP = 3
_GRID = 6
_ODIM = 128
_TN = 1024

_NC = 2
_NS = 16
_NW = _NC * _NS
_LANES = 16
_TB = 128
_OSTRIDE = 129
_REP = _NCOMP * _GRID * _ODIM + 1


def _stage_a_body(x_ref, p_ref, w_ref, off_ref, a_ref, b_ref):
    x = x_ref[...]
    projt = lax.dot_general(
        p_ref[...], x, (((0,), (1,)), ((), ())),
        preferred_element_type=jnp.float32,
    )
    p = jnp.clip(projt, -0.99, 0.99)
    xg = (p + 1.0) * (0.5 * (_GRID - 1))
    idx = jnp.clip(xg.astype(jnp.int32), 0, _GRID - 2)
    gs = idx.astype(jnp.float32) * (2.0 / (_GRID - 1)) - 1.0
    t = (p - gs) * (0.5 * (_GRID - 1))
    w = w_ref[...]
    comp = lax.broadcasted_iota(jnp.int32, idx.shape, 0)
    off_ref[...] = (comp * _GRID + idx) * _ODIM
    a_ref[...] = w * (1.0 - t)
    b_ref[...] = w * t


def _stage_a(x, pmat, w):
    n = x.shape[0]
    d = x.shape[1]
    shape = jax.ShapeDtypeStruct((_NCOMP, n), jnp.float32)
    return pl.pallas_call(
        _stage_a_body,
        grid=(n // _TN,),
        in_specs=[
            pl.BlockSpec((_TN, d), lambda i: (i, 0)),
            pl.BlockSpec((d, _NCOMP), lambda i: (0, 0)),
            pl.BlockSpec((_NCOMP, 1), lambda i: (0, 0)),
        ],
        out_specs=[
            pl.BlockSpec((_NCOMP, _TN), lambda i: (0, i)),
            pl.BlockSpec((_NCOMP, _TN), lambda i: (0, i)),
            pl.BlockSpec((_NCOMP, _TN), lambda i: (0, i)),
        ],
        out_shape=[
            jax.ShapeDtypeStruct((_NCOMP, n), jnp.int32),
            shape,
            shape,
        ],
        compiler_params=pltpu.CompilerParams(
            dimension_semantics=("arbitrary",)
        ),
    )(x, pmat, w)


def _tanh_sc(y):
    e = jnp.exp(-2.0 * jnp.abs(y))
    th = (1.0 - e) / (1.0 + e)
    return jnp.where(y < 0.0, -th, th)


def _stage_b_body(off_hbm, a_hbm, b_hbm, tbl_hbm, out_hbm,
                  offv, av, bv, tblv, outv):
    wid = lax.axis_index("s") * _NC + lax.axis_index("c")
    n = off_hbm.shape[0] // _NCOMP
    cn = n // _NW
    base = wid * cn
    for i in range(_NCOMP):
        pltpu.sync_copy(off_hbm.at[pl.ds(i * n + base, cn)],
                        offv.at[pl.ds(i * cn, cn)])
        pltpu.sync_copy(a_hbm.at[pl.ds(i * n + base, cn)],
                        av.at[pl.ds(i * cn, cn)])
        pltpu.sync_copy(b_hbm.at[pl.ds(i * n + base, cn)],
                        bv.at[pl.ds(i * cn, cn)])
    pltpu.sync_copy(tbl_hbm, tblv)

    nblk = cn // _TB
    ngrp = _TB // _LANES
    lane = lax.iota(jnp.int32, _LANES)
    lanebase = lane * _REP

    def grp_body(g, blk):
        gb = blk * _TB + g * _LANES
        offg = [offv[pl.ds(i * cn + gb, _LANES)] for i in range(_NCOMP)]
        ag = [av[pl.ds(i * cn + gb, _LANES)] for i in range(_NCOMP)]
        bg = [bv[pl.ds(i * cn + gb, _LANES)] for i in range(_NCOMP)]
        bases = [offg[i] + lanebase for i in range(_NCOMP)]
        rows = g * _LANES + lane

        def d_body(d8, carry):
            for du in range(8):
                d = d8 * 8 + du
                acc = None
                for i in range(_NCOMP):
                    p0 = plsc.load_gather(tblv, [bases[i] + d])
                    p1 = plsc.load_gather(tblv, [bases[i] + (_ODIM + d)])
                    contrib = ag[i] * p0 + bg[i] * p1
                    acc = contrib if i == 0 else acc + contrib
                cols = jnp.full((_LANES,), d, jnp.int32)
                plsc.store_scatter(outv, [rows, cols], _tanh_sc(acc))
            return carry

        lax.fori_loop(0, _ODIM // 8, d_body, 0)
        return blk

    def blk_body(blk, carry):
        lax.fori_loop(0, ngrp, grp_body, blk)
        pltpu.sync_copy(
            outv.at[:, pl.ds(0, _ODIM)],
            out_hbm.at[pl.ds(base + blk * _TB, _TB)],
        )
        return carry

    lax.fori_loop(0, nblk, blk_body, 0)


def _stage_b(offs, a, b, tbl):
    n = offs.shape[0] // _NCOMP
    cn = n // _NW
    body = pl.kernel(
        _stage_b_body,
        out_type=jax.ShapeDtypeStruct((n, _ODIM), jnp.float32),
        mesh=plsc.VectorSubcoreMesh(
            core_axis_name="c", subcore_axis_name="s"
        ),
        compiler_params=pltpu.CompilerParams(needs_layout_passes=False),
        scratch_types=[
            pltpu.VMEM((_NCOMP * cn,), jnp.int32),
            pltpu.VMEM((_NCOMP * cn,), jnp.float32),
            pltpu.VMEM((_NCOMP * cn,), jnp.float32),
            pltpu.VMEM((_LANES * _REP,), jnp.float32),
            pltpu.VMEM((_TB, _OSTRIDE), jnp.float32),
        ],
    )
    return body(offs, a, b, tbl)


def kernel(inputs, projections, control_points, component_weights):
    B, S, D = inputs.shape
    n = B * S
    x = inputs.reshape(n, D)
    pmat = projections[:, :, 0].T
    w = component_weights.reshape(_NCOMP, 1)
    flat = control_points.reshape(-1)
    rep = jnp.pad(flat, (0, _REP - flat.shape[0]))
    tbl = jnp.broadcast_to(rep, (_LANES, _REP)).reshape(-1)

    offs, a, b = _stage_a(x, pmat, w)
    out = _stage_b(offs.reshape(-1), a.reshape(-1), b.reshape(-1), tbl)
    return out.reshape(B, S, _ODIM)


# speedup vs baseline: 1.8817x; 1.3129x over previous
import jax
import jax.numpy as jnp
from jax import lax
from jax.experimental import pallas as pl
from jax.experimental.pallas import tpu as pltpu
from jax.experimental.pallas import tpu_sc as plsc

_NCOMP = 3
_GRID = 6
_ODIM = 128
_TN = 1024

_NC = 2
_NS = 16
_NW = _NC * _NS
_LANES = 16
_TB = 128


def _stage_a_body(x_ref, p_ref, w_ref, off_ref, a_ref, b_ref):
    x = x_ref[...]
    projt = lax.dot_general(
        p_ref[...], x, (((0,), (1,)), ((), ())),
        preferred_element_type=jnp.float32,
    )
    p = jnp.clip(projt, -0.99, 0.99)
    xg = (p + 1.0) * (0.5 * (_GRID - 1))
    idx = jnp.clip(xg.astype(jnp.int32), 0, _GRID - 2)
    gs = idx.astype(jnp.float32) * (2.0 / (_GRID - 1)) - 1.0
    t = (p - gs) * (0.5 * (_GRID - 1))
    w = w_ref[...]
    comp = lax.broadcasted_iota(jnp.int32, idx.shape, 0)
    off_ref[...] = (comp * _GRID + idx) * _ODIM
    a_ref[...] = w * (1.0 - t)
    b_ref[...] = w * t


def _stage_a(x, pmat, w):
    n = x.shape[0]
    d = x.shape[1]
    shape = jax.ShapeDtypeStruct((_NCOMP, n), jnp.float32)
    return pl.pallas_call(
        _stage_a_body,
        grid=(n // _TN,),
        in_specs=[
            pl.BlockSpec((_TN, d), lambda i: (i, 0)),
            pl.BlockSpec((d, _NCOMP), lambda i: (0, 0)),
            pl.BlockSpec((_NCOMP, 1), lambda i: (0, 0)),
        ],
        out_specs=[
            pl.BlockSpec((_NCOMP, _TN), lambda i: (0, i)),
            pl.BlockSpec((_NCOMP, _TN), lambda i: (0, i)),
            pl.BlockSpec((_NCOMP, _TN), lambda i: (0, i)),
        ],
        out_shape=[
            jax.ShapeDtypeStruct((_NCOMP, n), jnp.int32),
            shape,
            shape,
        ],
        compiler_params=pltpu.CompilerParams(
            dimension_semantics=("arbitrary",)
        ),
    )(x, pmat, w)


def _tanh_sc(y):
    e = jnp.exp(y + y)
    return 1.0 - 2.0 / (e + 1.0)


def _stage_b_body(off_hbm, a_hbm, b_hbm, tbl_hbm, out_hbm,
                  offv, av, bv, tblv, outv):
    wid = lax.axis_index("s") * _NC + lax.axis_index("c")
    n = off_hbm.shape[0] // _NCOMP
    cn = n // _NW
    base = wid * cn
    for i in range(_NCOMP):
        pltpu.sync_copy(off_hbm.at[pl.ds(i * n + base, cn)],
                        offv.at[pl.ds(i * cn, cn)])
        pltpu.sync_copy(a_hbm.at[pl.ds(i * n + base, cn)],
                        av.at[pl.ds(i * cn, cn)])
        pltpu.sync_copy(b_hbm.at[pl.ds(i * n + base, cn)],
                        bv.at[pl.ds(i * cn, cn)])
    pltpu.sync_copy(tbl_hbm, tblv)

    nblk = cn // _TB
    ngrp = _TB // _LANES

    def grp_body(g, blk):
        gb = blk * _TB + g * _LANES
        offg = [offv[pl.ds(i * cn + gb, _LANES)] for i in range(_NCOMP)]
        ag = [av[pl.ds(i * cn + gb, _LANES)] for i in range(_NCOMP)]
        bg = [bv[pl.ds(i * cn + gb, _LANES)] for i in range(_NCOMP)]
        for t in range(_LANES):
            acc = [None] * (_ODIM // _LANES)
            for i in range(_NCOMP):
                off = offg[i][t]
                ai = ag[i][t]
                bi = bg[i][t]
                for dv in range(_ODIM // _LANES):
                    p0 = tblv[pl.ds(off + dv * _LANES, _LANES)]
                    p1 = tblv[pl.ds(off + _ODIM + dv * _LANES, _LANES)]
                    contrib = ai * p0 + bi * p1
                    acc[dv] = contrib if i == 0 else acc[dv] + contrib
            ob = (g * _LANES + t) * _ODIM
            for dv in range(_ODIM // _LANES):
                outv[pl.ds(ob + dv * _LANES, _LANES)] = _tanh_sc(acc[dv])
        return blk

    def blk_body(blk, carry):
        lax.fori_loop(0, ngrp, grp_body, blk)
        pltpu.sync_copy(
            outv,
            out_hbm.at[pl.ds((base + blk * _TB) * _ODIM, _TB * _ODIM)],
        )
        return carry

    lax.fori_loop(0, nblk, blk_body, 0)


def _stage_b(offs, a, b, tbl):
    n = offs.shape[0] // _NCOMP
    cn = n // _NW
    body = pl.kernel(
        _stage_b_body,
        out_type=jax.ShapeDtypeStruct((n * _ODIM,), jnp.float32),
        mesh=plsc.VectorSubcoreMesh(
            core_axis_name="c", subcore_axis_name="s"
        ),
        scratch_types=[
            pltpu.VMEM((_NCOMP * cn,), jnp.int32),
            pltpu.VMEM((_NCOMP * cn,), jnp.float32),
            pltpu.VMEM((_NCOMP * cn,), jnp.float32),
            pltpu.VMEM((_NCOMP * _GRID * _ODIM,), jnp.float32),
            pltpu.VMEM((_TB * _ODIM,), jnp.float32),
        ],
    )
    return body(offs, a, b, tbl)


def kernel(inputs, projections, control_points, component_weights):
    B, S, D = inputs.shape
    n = B * S
    x = inputs.reshape(n, D)
    pmat = projections[:, :, 0].T
    w = component_weights.reshape(_NCOMP, 1)
    tbl = control_points.reshape(_NCOMP * _GRID * _ODIM)

    offs, a, b = _stage_a(x, pmat, w)
    out = _stage_b(offs.reshape(-1), a.reshape(-1), b.reshape(-1), tbl)
    return out.reshape(B, S, _ODIM)
